# baseline (speedup 1.0000x reference)
import functools

import jax
import jax.numpy as jnp
from jax.experimental import pallas as pl
from jax.experimental.pallas import tpu as pltpu

_N_CB = 8
_K = 1024
_CD = 64
_NW = 3 * _CD + 3
_NSPLIT = 4


def _rvq_body(z_ref, cb_ref, zq_ref, idx_ref, logits_ref,
              cbh_ref, w_ref, cbn_ref):
    blk = z_ref.shape[1]
    hb = blk // _NSPLIT

    first = jnp.logical_and(pl.program_id(0) == 0, pl.program_id(1) == 0)

    @pl.when(first)
    def _init():
        cb = cb_ref[...]
        hi = cb.astype(jnp.bfloat16)
        r1 = cb - hi.astype(jnp.float32)
        mid = r1.astype(jnp.bfloat16)
        r2 = r1 - mid.astype(jnp.float32)
        lo = r2.astype(jnp.bfloat16)
        cbh_ref[...] = hi
        k = jax.lax.broadcasted_iota(jnp.int32, (_N_CB, _K, 1), 1)
        ihi = ((k // 256) * 256).astype(jnp.bfloat16)
        ilo = (k % 256).astype(jnp.bfloat16)
        ones = jnp.ones((_N_CB, _K, 1), jnp.bfloat16)
        w_ref[...] = jnp.concatenate([hi, mid, lo, ihi, ilo, ones], axis=-1)
        for j in range(_N_CB):
            cbj = cb_ref[j]
            cbn_ref[j] = jnp.sum(cbj * cbj, axis=-1)

    carries = [None] * _NSPLIT
    ok = None
    for i in range(_N_CB):
        for s in range(_NSPLIT):
            rows = pl.ds(s * hb, hb)
            chunk = z_ref[0, rows, i * _CD:(i + 1) * _CD]
            res = chunk if i == 0 else chunk + carries[s]
            cross2 = jax.lax.dot_general(
                (res + res).astype(jnp.bfloat16), cbh_ref[i],
                (((1,), (1,)), ((), ())),
                preferred_element_type=jnp.float32)
            negdist = cross2 - (jnp.sum(res * res, axis=1, keepdims=True)
                                + cbn_ref[i][None, :])
            logits_ref[0, i, rows] = negdist
            m = jnp.max(negdist, axis=-1, keepdims=True)
            eqmask = (negdist == m).astype(jnp.bfloat16)
            g = jax.lax.dot_general(
                eqmask, w_ref[i], (((1,), (0,)), ((), ())),
                preferred_element_type=jnp.float32)
            q = (g[:, 0:_CD] + g[:, _CD:2 * _CD]) + g[:, 2 * _CD:3 * _CD]
            idxf = g[:, 3 * _CD] + g[:, 3 * _CD + 1]
            cnt = g[:, 3 * _CD + 2]
            idx_ref[0, i, rows] = idxf.astype(jnp.int32)
            zq_ref[0, rows, i * _CD:(i + 1) * _CD] = q
            carries[s] = res - q
            ok_i = jnp.all(cnt == 1.0)
            ok = ok_i if ok is None else jnp.logical_and(ok, ok_i)

    @pl.when(jnp.logical_not(ok))
    def _tie_fallback():
        res = z_ref[0, :, 0:_CD]
        for i in range(_N_CB):
            if i > 0:
                res = z_ref[0, :, i * _CD:(i + 1) * _CD] + res
            cross2 = jax.lax.dot_general(
                (res + res).astype(jnp.bfloat16), cbh_ref[i],
                (((1,), (1,)), ((), ())),
                preferred_element_type=jnp.float32)
            negdist = cross2 - (jnp.sum(res * res, axis=1, keepdims=True)
                                + cbn_ref[i][None, :])
            logits_ref[0, i] = negdist
            idx = jnp.argmax(negdist, axis=-1).astype(jnp.int32)
            onehot = (jax.lax.broadcasted_iota(jnp.int32, (blk, _K), 1)
                      == idx[:, None]).astype(jnp.bfloat16)
            g2 = jax.lax.dot_general(
                onehot, w_ref[i], (((1,), (0,)), ((), ())),
                preferred_element_type=jnp.float32)
            q2 = (g2[:, 0:_CD] + g2[:, _CD:2 * _CD]) + g2[:, 2 * _CD:3 * _CD]
            idx_ref[0, i] = idx
            zq_ref[0, :, i * _CD:(i + 1) * _CD] = q2
            res = res - q2


@functools.partial(jax.jit, static_argnames=("blk",))
def _rvq(z, codebooks, blk):
    Bb, Tt, Dd = z.shape
    grid = (Bb, Tt // blk)
    out_shapes = (
        jax.ShapeDtypeStruct((Bb, Tt, Dd), jnp.float32),
        jax.ShapeDtypeStruct((Bb, _N_CB, Tt), jnp.int32),
        jax.ShapeDtypeStruct((Bb, _N_CB, Tt, _K), jnp.float32),
    )
    z_spec = pl.BlockSpec((1, blk, Dd), lambda b, t: (b, t, 0))
    cb_spec = pl.BlockSpec((_N_CB, _K, _CD), lambda b, t: (0, 0, 0))
    zq_spec = pl.BlockSpec((1, blk, Dd), lambda b, t: (b, t, 0))
    idx_spec = pl.BlockSpec((1, _N_CB, blk), lambda b, t: (b, 0, t))
    logits_spec = pl.BlockSpec((1, _N_CB, blk, _K), lambda b, t: (b, 0, t, 0))
    return pl.pallas_call(
        _rvq_body,
        grid=grid,
        in_specs=[z_spec, cb_spec],
        out_specs=(zq_spec, idx_spec, logits_spec),
        out_shape=out_shapes,
        scratch_shapes=[
            pltpu.VMEM((_N_CB, _K, _CD), jnp.bfloat16),
            pltpu.VMEM((_N_CB, _K, _NW), jnp.bfloat16),
            pltpu.VMEM((_N_CB, _K), jnp.float32),
        ],
        compiler_params=pltpu.CompilerParams(
            dimension_semantics=("arbitrary", "arbitrary")),
    )(z, codebooks)


def kernel(z, codebooks):
    return _rvq(z, codebooks, blk=256)


# speedup vs baseline: 1.7654x; 1.7654x over previous
import functools

import jax
import jax.numpy as jnp
from jax.experimental import pallas as pl
from jax.experimental.pallas import tpu as pltpu

_N_CB = 8
_K = 1024
_CD = 64
_NW = 3 * _CD + 3
_NSPLIT = 2


def _rvq_body(z_ref, cb_ref, zq_ref, idx_ref, logits_ref,
              cbh_ref, w_ref, cbn_ref):
    blk = z_ref.shape[1]
    hb = blk // _NSPLIT

    first = jnp.logical_and(pl.program_id(0) == 0, pl.program_id(1) == 0)

    @pl.when(first)
    def _init():
        cb = cb_ref[...]
        hi = cb.astype(jnp.bfloat16)
        r1 = cb - hi.astype(jnp.float32)
        mid = r1.astype(jnp.bfloat16)
        r2 = r1 - mid.astype(jnp.float32)
        lo = r2.astype(jnp.bfloat16)
        cbh_ref[...] = hi
        k = jax.lax.broadcasted_iota(jnp.int32, (_N_CB, _K, 1), 1)
        ihi = ((k // 256) * 256).astype(jnp.bfloat16)
        ilo = (k % 256).astype(jnp.bfloat16)
        ones = jnp.ones((_N_CB, _K, 1), jnp.bfloat16)
        w_ref[...] = jnp.concatenate([hi, mid, lo, ihi, ilo, ones], axis=-1)
        for j in range(_N_CB):
            cbj = cb_ref[j]
            cbn_ref[j] = jnp.sum(cbj * cbj, axis=-1)

    carries = [None] * _NSPLIT
    ok = None
    for i in range(_N_CB):
        for s in range(_NSPLIT):
            rows = pl.ds(s * hb, hb)
            chunk = z_ref[0, rows, i * _CD:(i + 1) * _CD]
            res = chunk if i == 0 else chunk + carries[s]
            cross2 = jax.lax.dot_general(
                (res + res).astype(jnp.bfloat16), cbh_ref[i],
                (((1,), (1,)), ((), ())),
                preferred_element_type=jnp.float32)
            negdist = cross2 - (jnp.sum(res * res, axis=1, keepdims=True)
                                + cbn_ref[i][None, :])
            logits_ref[0, i, rows] = negdist
            m = jnp.max(negdist, axis=-1, keepdims=True)
            eqmask = (negdist == m).astype(jnp.bfloat16)
            g = jax.lax.dot_general(
                eqmask, w_ref[i], (((1,), (0,)), ((), ())),
                preferred_element_type=jnp.float32)
            q = (g[:, 0:_CD] + g[:, _CD:2 * _CD]) + g[:, 2 * _CD:3 * _CD]
            idxf = g[:, 3 * _CD] + g[:, 3 * _CD + 1]
            cnt = g[:, 3 * _CD + 2]
            idx_ref[0, i, rows] = idxf.astype(jnp.int32)
            zq_ref[0, rows, i * _CD:(i + 1) * _CD] = q
            carries[s] = res - q
            ok_i = jnp.all(cnt == 1.0)
            ok = ok_i if ok is None else jnp.logical_and(ok, ok_i)

    @pl.when(jnp.logical_not(ok))
    def _tie_fallback():
        res = z_ref[0, :, 0:_CD]
        for i in range(_N_CB):
            if i > 0:
                res = z_ref[0, :, i * _CD:(i + 1) * _CD] + res
            cross2 = jax.lax.dot_general(
                (res + res).astype(jnp.bfloat16), cbh_ref[i],
                (((1,), (1,)), ((), ())),
                preferred_element_type=jnp.float32)
            negdist = cross2 - (jnp.sum(res * res, axis=1, keepdims=True)
                                + cbn_ref[i][None, :])
            logits_ref[0, i] = negdist
            idx = jnp.argmax(negdist, axis=-1).astype(jnp.int32)
            onehot = (jax.lax.broadcasted_iota(jnp.int32, (blk, _K), 1)
                      == idx[:, None]).astype(jnp.bfloat16)
            g2 = jax.lax.dot_general(
                onehot, w_ref[i], (((1,), (0,)), ((), ())),
                preferred_element_type=jnp.float32)
            q2 = (g2[:, 0:_CD] + g2[:, _CD:2 * _CD]) + g2[:, 2 * _CD:3 * _CD]
            idx_ref[0, i] = idx
            zq_ref[0, :, i * _CD:(i + 1) * _CD] = q2
            res = res - q2


@functools.partial(jax.jit, static_argnames=("blk",))
def _rvq(z, codebooks, blk):
    Bb, Tt, Dd = z.shape
    grid = (Bb, Tt // blk)
    out_shapes = (
        jax.ShapeDtypeStruct((Bb, Tt, Dd), jnp.float32),
        jax.ShapeDtypeStruct((Bb, _N_CB, Tt), jnp.int32),
        jax.ShapeDtypeStruct((Bb, _N_CB, Tt, _K), jnp.float32),
    )
    z_spec = pl.BlockSpec((1, blk, Dd), lambda b, t: (b, t, 0))
    cb_spec = pl.BlockSpec((_N_CB, _K, _CD), lambda b, t: (0, 0, 0))
    zq_spec = pl.BlockSpec((1, blk, Dd), lambda b, t: (b, t, 0))
    idx_spec = pl.BlockSpec((1, _N_CB, blk), lambda b, t: (b, 0, t))
    logits_spec = pl.BlockSpec((1, _N_CB, blk, _K), lambda b, t: (b, 0, t, 0))
    return pl.pallas_call(
        _rvq_body,
        grid=grid,
        in_specs=[z_spec, cb_spec],
        out_specs=(zq_spec, idx_spec, logits_spec),
        out_shape=out_shapes,
        scratch_shapes=[
            pltpu.VMEM((_N_CB, _K, _CD), jnp.bfloat16),
            pltpu.VMEM((_N_CB, _K, _NW), jnp.bfloat16),
            pltpu.VMEM((_N_CB, _K), jnp.float32),
        ],
        compiler_params=pltpu.CompilerParams(
            dimension_semantics=("arbitrary", "arbitrary")),
    )(z, codebooks)


def kernel(z, codebooks):
    return _rvq(z, codebooks, blk=512)
